# baseline (speedup 1.0000x reference)
import functools

import jax
import jax.numpy as jnp
from jax import lax
from jax.experimental import pallas as pl
from jax.experimental.pallas import tpu as pltpu
from jax.experimental.pallas import tpu_sc as plsc

N = 10000
E = 320000
D = 128
L = 3

NC = 2
NS = 16
NW = NC * NS
CH = 128
K = 80
E_PAD = NW * K * CH
NPAD = 10240
ZROWS = NPAD // NS
OROWS = NPAD // NS

@functools.cache
def _build_sc_segment_sum():
    mesh = plsc.VectorSubcoreMesh(core_axis_name="c", subcore_axis_name="s")

    @functools.partial(
        pl.kernel,
        out_type=jax.ShapeDtypeStruct((NC, NPAD, D), jnp.float32),
        mesh=mesh,
        scratch_types=[
            pltpu.VMEM((K // 2, CH), jnp.int32),
            pltpu.VMEM((K // 2, CH), jnp.int32),
            pltpu.VMEM((CH, D), jnp.float32),
            pltpu.VMEM((CH, D), jnp.float32),
            pltpu.VMEM_SHARED((NPAD, D), jnp.float32),
            pltpu.SemaphoreType.DMA,
            pltpu.SemaphoreType.DMA,
            pltpu.SemaphoreType.DMA,
            pltpu.SemaphoreType.DMA,
        ],
    )
    def sc_segment_sum(h_hbm, src_hbm, dst_hbm, zeros_hbm, out_hbm,
                       src_v, dst_v, rows0_v, rows1_v, acc_sh,
                       g0, g1, s0, s1):
        c = lax.axis_index("c")
        s = lax.axis_index("s")
        wid = s * NC + c
        pltpu.sync_copy(zeros_hbm.at[pl.ds(s * ZROWS, ZROWS)],
                        acc_sh.at[pl.ds(s * ZROWS, ZROWS)])
        plsc.subcore_barrier()

        KH = K // 2
        for half in range(2):
            pltpu.sync_copy(src_hbm.at[wid, pl.ds(half * KH, KH)], src_v)
            pltpu.sync_copy(dst_hbm.at[wid, pl.ds(half * KH, KH)], dst_v)

            pltpu.async_copy(h_hbm.at[src_v.at[0]], rows0_v, g0)
            pltpu.async_copy(h_hbm.at[src_v.at[1]], rows1_v, g1)

            def body(i, carry):
                j = 2 * i
                jn0 = jnp.minimum(j + 2, KH - 1)
                jn1 = jnp.minimum(j + 3, KH - 1)
                pltpu.make_async_copy(
                    h_hbm.at[src_v.at[j]], rows0_v, g0).wait()
                d0 = pltpu.async_copy(
                    rows0_v, acc_sh.at[dst_v.at[j]], s0, add=True)
                pltpu.make_async_copy(
                    h_hbm.at[src_v.at[j + 1]], rows1_v, g1).wait()
                d1 = pltpu.async_copy(
                    rows1_v, acc_sh.at[dst_v.at[j + 1]], s1, add=True)
                d0.wait()
                pltpu.async_copy(h_hbm.at[src_v.at[jn0]], rows0_v, g0)
                d1.wait()
                pltpu.async_copy(h_hbm.at[src_v.at[jn1]], rows1_v, g1)
                return carry

            lax.fori_loop(0, KH // 2, body, 0)
            pltpu.make_async_copy(
                h_hbm.at[src_v.at[KH - 1]], rows0_v, g0).wait()
            pltpu.make_async_copy(
                h_hbm.at[src_v.at[KH - 1]], rows1_v, g1).wait()

        plsc.subcore_barrier()
        pltpu.sync_copy(acc_sh.at[pl.ds(s * OROWS, OROWS)],
                        out_hbm.at[c, pl.ds(s * OROWS, OROWS)])

    return sc_segment_sum


def _tc_dense(h_ref, agg_ref, w_ref, b_ref, eps_ref,
              g1_ref, b1_ref, g2_ref, b2_ref, out_ref):
    agg = agg_ref[0, 0:N, :] + agg_ref[1, 0:N, :]
    u = (1.0 + eps_ref[0, 0]) * h_ref[...] + agg
    y = jnp.dot(u, w_ref[...], preferred_element_type=jnp.float32) + b_ref[...]
    m = jnp.mean(y, axis=0, keepdims=True)
    v = jnp.mean((y - m) ** 2, axis=0, keepdims=True)
    y = (y - m) * lax.rsqrt(v + 1e-5) * g1_ref[...] + b1_ref[...]
    y = jnp.maximum(y, 0.0)
    m2 = jnp.mean(y, axis=0, keepdims=True)
    v2 = jnp.mean((y - m2) ** 2, axis=0, keepdims=True)
    y = (y - m2) * lax.rsqrt(v2 + 1e-5) * g2_ref[...] + b2_ref[...]
    out_ref[...] = jnp.maximum(y, 0.0)


_tc_call = pl.pallas_call(
    _tc_dense,
    out_shape=jax.ShapeDtypeStruct((N, D), jnp.float32),
)


def kernel(x, edge_index, W, b, eps, gamma1, beta1, gamma2, beta2):
    src = edge_index[0]
    dst = edge_index[1]
    pad = E_PAD - E
    r = jnp.arange(pad, dtype=jnp.int32)
    src_p = jnp.concatenate([src, r % N]).reshape(NW, K, CH)
    dst_p = jnp.concatenate([dst, N + r % (NPAD - N)]).reshape(NW, K, CH)
    zeros = jnp.zeros((NPAD, D), jnp.float32)

    sc_segment_sum = _build_sc_segment_sum()
    h = x
    for i in range(L):
        agg = sc_segment_sum(h, src_p, dst_p, zeros)
        h = _tc_call(h, agg, W[i], b[i].reshape(1, D),
                     eps[i].reshape(1, 1),
                     gamma1[i].reshape(1, D), beta1[i].reshape(1, D),
                     gamma2[i].reshape(1, D), beta2[i].reshape(1, D))
    return h


# speedup vs baseline: 1.2937x; 1.2937x over previous
import functools

import jax
import jax.numpy as jnp
from jax import lax
from jax.experimental import pallas as pl
from jax.experimental.pallas import tpu as pltpu
from jax.experimental.pallas import tpu_sc as plsc

N = 10000
E = 320000
D = 128
L = 3

NC = 2
NS = 16
NW = NC * NS
CH = 128
K = 80
E_PAD = NW * K * CH
NPAD = 10240
ZROWS = NPAD // NS
OROWS = NPAD // NS

@functools.cache
def _build_sc_segment_sum():
    mesh = plsc.VectorSubcoreMesh(core_axis_name="c", subcore_axis_name="s")

    @functools.partial(
        pl.kernel,
        out_type=jax.ShapeDtypeStruct((NC, NPAD, D), jnp.float32),
        mesh=mesh,
        scratch_types=[
            pltpu.VMEM((K // 2, CH), jnp.int32),
            pltpu.VMEM((K // 2, CH), jnp.int32),
            pltpu.VMEM((CH, D), jnp.float32),
            pltpu.VMEM((CH, D), jnp.float32),
            pltpu.VMEM_SHARED((NPAD, D), jnp.float32),
            pltpu.SemaphoreType.DMA,
            pltpu.SemaphoreType.DMA,
        ],
    )
    def sc_segment_sum(h_hbm, src_hbm, dst_hbm, zeros_hbm, out_hbm,
                       src_v, dst_v, rows0_v, rows1_v, acc_sh, g0, g1):
        c = lax.axis_index("c")
        s = lax.axis_index("s")
        wid = s * NC + c

        KH = K // 2
        for half in range(2):
            pltpu.sync_copy(src_hbm.at[wid, pl.ds(half * KH, KH)], src_v)
            pltpu.sync_copy(dst_hbm.at[wid, pl.ds(half * KH, KH)], dst_v)

            pltpu.async_copy(h_hbm.at[src_v.at[0]], rows0_v, g0)
            pltpu.async_copy(h_hbm.at[src_v.at[1]], rows1_v, g1)

            if half == 0:
                pltpu.sync_copy(zeros_hbm.at[pl.ds(s * ZROWS, ZROWS)],
                                acc_sh.at[pl.ds(s * ZROWS, ZROWS)])
                plsc.subcore_barrier()

            def body(i, carry):
                j = 2 * i
                jn0 = jnp.minimum(j + 2, KH - 1)
                jn1 = jnp.minimum(j + 3, KH - 1)
                pltpu.make_async_copy(
                    h_hbm.at[src_v.at[j]], rows0_v, g0).wait()
                pltpu.sync_copy(rows0_v, acc_sh.at[dst_v.at[j]], add=True)
                pltpu.async_copy(h_hbm.at[src_v.at[jn0]], rows0_v, g0)
                pltpu.make_async_copy(
                    h_hbm.at[src_v.at[j + 1]], rows1_v, g1).wait()
                pltpu.sync_copy(rows1_v, acc_sh.at[dst_v.at[j + 1]],
                                add=True)
                pltpu.async_copy(h_hbm.at[src_v.at[jn1]], rows1_v, g1)
                return carry

            lax.fori_loop(0, KH // 2, body, 0)
            pltpu.make_async_copy(
                h_hbm.at[src_v.at[KH - 1]], rows0_v, g0).wait()
            pltpu.make_async_copy(
                h_hbm.at[src_v.at[KH - 1]], rows1_v, g1).wait()

        plsc.subcore_barrier()
        pltpu.sync_copy(acc_sh.at[pl.ds(s * OROWS, OROWS)],
                        out_hbm.at[c, pl.ds(s * OROWS, OROWS)])

    return sc_segment_sum


def _tc_dense(h_ref, agg_ref, w_ref, b_ref, eps_ref,
              g1_ref, b1_ref, g2_ref, b2_ref, out_ref):
    agg = agg_ref[0, 0:N, :] + agg_ref[1, 0:N, :]
    u = (1.0 + eps_ref[0, 0]) * h_ref[...] + agg
    y = jnp.dot(u, w_ref[...], preferred_element_type=jnp.float32) + b_ref[...]
    m = jnp.mean(y, axis=0, keepdims=True)
    v = jnp.mean((y - m) ** 2, axis=0, keepdims=True)
    y = (y - m) * lax.rsqrt(v + 1e-5) * g1_ref[...] + b1_ref[...]
    y = jnp.maximum(y, 0.0)
    m2 = jnp.mean(y, axis=0, keepdims=True)
    v2 = jnp.mean((y - m2) ** 2, axis=0, keepdims=True)
    y = (y - m2) * lax.rsqrt(v2 + 1e-5) * g2_ref[...] + b2_ref[...]
    out_ref[...] = jnp.maximum(y, 0.0)


_tc_call = pl.pallas_call(
    _tc_dense,
    out_shape=jax.ShapeDtypeStruct((N, D), jnp.float32),
)


def kernel(x, edge_index, W, b, eps, gamma1, beta1, gamma2, beta2):
    src = edge_index[0]
    dst = edge_index[1]
    pad = E_PAD - E
    r = jnp.arange(pad, dtype=jnp.int32)
    src_p = jnp.concatenate([src, r % N]).reshape(NW, K, CH)
    dst_p = jnp.concatenate([dst, N + r % (NPAD - N)]).reshape(NW, K, CH)
    zeros = jnp.zeros((NPAD, D), jnp.float32)

    sc_segment_sum = _build_sc_segment_sum()
    h = x
    for i in range(L):
        agg = sc_segment_sum(h, src_p, dst_p, zeros)
        h = _tc_call(h, agg, W[i], b[i].reshape(1, D),
                     eps[i].reshape(1, 1),
                     gamma1[i].reshape(1, D), beta1[i].reshape(1, D),
                     gamma2[i].reshape(1, D), beta2[i].reshape(1, D))
    return h


# speedup vs baseline: 1.3201x; 1.0204x over previous
import functools

import jax
import jax.numpy as jnp
from jax import lax
from jax.experimental import pallas as pl
from jax.experimental.pallas import tpu as pltpu
from jax.experimental.pallas import tpu_sc as plsc

N = 10000
E = 320000
D = 128
L = 3

NC = 2
NS = 16
NW = NC * NS
CH = 128
K = 80
E_PAD = NW * K * CH
NPAD = 10240
ZROWS = NPAD // NS
OROWS = NPAD // NS

@functools.cache
def _build_sc_segment_sum():
    mesh = plsc.VectorSubcoreMesh(core_axis_name="c", subcore_axis_name="s")

    @functools.partial(
        pl.kernel,
        out_type=jax.ShapeDtypeStruct((NC, NPAD, D), jnp.float32),
        mesh=mesh,
        scratch_types=[
            pltpu.VMEM((K // 2, CH), jnp.int32),
            pltpu.VMEM((K // 2, CH), jnp.int32),
            pltpu.VMEM((CH, D), jnp.float32),
            pltpu.VMEM((CH, D), jnp.float32),
            pltpu.VMEM_SHARED((NPAD, D), jnp.float32),
            pltpu.SemaphoreType.DMA,
            pltpu.SemaphoreType.DMA,
        ],
    )
    def sc_segment_sum(h_hbm, src_hbm, dst_hbm, zeros_hbm, out_hbm,
                       src_v, dst_v, rows0_v, rows1_v, acc_sh, g0, g1):
        c = lax.axis_index("c")
        s = lax.axis_index("s")
        wid = s * NC + c

        KH = K // 2
        for half in range(2):
            pltpu.sync_copy(src_hbm.at[wid, pl.ds(half * KH, KH)], src_v)
            pltpu.sync_copy(dst_hbm.at[wid, pl.ds(half * KH, KH)], dst_v)

            pltpu.async_copy(h_hbm.at[src_v.at[0]], rows0_v, g0)
            pltpu.async_copy(h_hbm.at[src_v.at[1]], rows1_v, g1)

            if half == 0:
                pltpu.sync_copy(zeros_hbm.at[pl.ds(s * ZROWS, ZROWS)],
                                acc_sh.at[pl.ds(s * ZROWS, ZROWS)])
                plsc.subcore_barrier()

            def body(i, carry):
                j = 2 * i
                jn0 = jnp.minimum(j + 2, KH - 1)
                jn1 = jnp.minimum(j + 3, KH - 1)
                pltpu.make_async_copy(
                    h_hbm.at[src_v.at[j]], rows0_v, g0).wait()
                pltpu.sync_copy(rows0_v, acc_sh.at[dst_v.at[j]], add=True)
                pltpu.async_copy(h_hbm.at[src_v.at[jn0]], rows0_v, g0)
                pltpu.make_async_copy(
                    h_hbm.at[src_v.at[j + 1]], rows1_v, g1).wait()
                pltpu.sync_copy(rows1_v, acc_sh.at[dst_v.at[j + 1]],
                                add=True)
                pltpu.async_copy(h_hbm.at[src_v.at[jn1]], rows1_v, g1)
                return carry

            lax.fori_loop(0, KH // 2, body, 0)
            pltpu.make_async_copy(
                h_hbm.at[src_v.at[KH - 1]], rows0_v, g0).wait()
            pltpu.make_async_copy(
                h_hbm.at[src_v.at[KH - 1]], rows1_v, g1).wait()

        plsc.subcore_barrier()
        pltpu.sync_copy(acc_sh.at[pl.ds(s * OROWS, OROWS)],
                        out_hbm.at[c, pl.ds(s * OROWS, OROWS)])

    return sc_segment_sum


def _tc_dense(h_ref, agg_ref, w_ref, b_ref, eps_ref,
              g1_ref, b1_ref, g2_ref, b2_ref, out_ref):
    agg = agg_ref[0, 0:N, :] + agg_ref[1, 0:N, :]
    u = (1.0 + eps_ref[0, 0]) * h_ref[...] + agg
    y = jnp.dot(u, w_ref[...], preferred_element_type=jnp.float32) + b_ref[...]
    m = jnp.mean(y, axis=0, keepdims=True)
    v = jnp.mean(y * y, axis=0, keepdims=True) - m * m
    k = lax.rsqrt(v + 1e-5) * g1_ref[...]
    y = jnp.maximum(y * k + (b1_ref[...] - m * k), 0.0)
    m2 = jnp.mean(y, axis=0, keepdims=True)
    v2 = jnp.mean(y * y, axis=0, keepdims=True) - m2 * m2
    k2 = lax.rsqrt(v2 + 1e-5) * g2_ref[...]
    out_ref[...] = jnp.maximum(y * k2 + (b2_ref[...] - m2 * k2), 0.0)


_tc_call = pl.pallas_call(
    _tc_dense,
    out_shape=jax.ShapeDtypeStruct((N, D), jnp.float32),
)


def kernel(x, edge_index, W, b, eps, gamma1, beta1, gamma2, beta2):
    src = edge_index[0]
    dst = edge_index[1]
    pad = E_PAD - E
    r = jnp.arange(pad, dtype=jnp.int32)
    src_p = jnp.concatenate([src, r % N]).reshape(NW, K, CH)
    dst_p = jnp.concatenate([dst, N + r % (NPAD - N)]).reshape(NW, K, CH)
    zeros = jnp.zeros((NPAD, D), jnp.float32)

    sc_segment_sum = _build_sc_segment_sum()
    h = x
    for i in range(L):
        agg = sc_segment_sum(h, src_p, dst_p, zeros)
        h = _tc_call(h, agg, W[i], b[i].reshape(1, D),
                     eps[i].reshape(1, 1),
                     gamma1[i].reshape(1, D), beta1[i].reshape(1, D),
                     gamma2[i].reshape(1, D), beta2[i].reshape(1, D))
    return h
